# baseline (speedup 1.0000x reference)
import functools

import jax
import jax.numpy as jnp
from jax import lax
from jax.experimental import pallas as pl
from jax.experimental.pallas import tpu as pltpu
from jax.experimental.pallas import tpu_sc as plsc

_MAXLEN = 200
_EMBED = 32
_NW = 32
_STRIPE = 128
_VOCAB = 1_000_000
_NLINE = _VOCAB // 4
_FULL_TILES = _VOCAB // 128
_MAIN_PW = _FULL_TILES // _NW

_PARAMS = pltpu.CompilerParams(
    use_tc_tiling_on_sc=True, needs_layout_passes=False)
_MESH = plsc.VectorSubcoreMesh(core_axis_name="c", subcore_axis_name="s")


def _wid():
    return lax.axis_index("s") * 2 + lax.axis_index("c")


@functools.partial(
    pl.kernel,
    mesh=_MESH,
    out_type=jax.ShapeDtypeStruct((_NLINE, 128), jnp.float32),
    scratch_types=[
        pltpu.VMEM((2, _EMBED, 128), jnp.float32),
        pltpu.VMEM((2, _EMBED, 128), jnp.float32),
        pltpu.SemaphoreType.DMA,
        pltpu.SemaphoreType.DMA,
        pltpu.SemaphoreType.DMA,
        pltpu.SemaphoreType.DMA,
    ],
    compiler_params=_PARAMS,
)
def _pack_table(tokt_hbm, tail_hbm, out_hbm, in_v, out_v, gs0, gs1, ss0, ss1):
    w = _wid()
    lanes = lax.iota(jnp.int32, 16)
    gsem = (gs0, gs1)
    ssem = (ss0, ss1)

    def fire_in(b, sl):
        pltpu.async_copy(
            tokt_hbm.at[:, pl.ds(b * 128, 128)], in_v.at[sl], gsem[sl])

    def wait_in(b, sl):
        pltpu.make_async_copy(
            tokt_hbm.at[:, pl.ds(b * 128, 128)], in_v.at[sl], gsem[sl]).wait()

    def fire_out(b, sl):
        pltpu.async_copy(
            out_v.at[sl], out_hbm.at[pl.ds(b * 32, 32)], ssem[sl])

    def wait_out(b, sl):
        pltpu.make_async_copy(
            out_v.at[sl], out_hbm.at[pl.ds(b * 32, 32)], ssem[sl]).wait()

    rows_l = [lax.shift_right_logical(lanes + 16 * lg, 2) for lg in range(8)]
    rot_l = [lax.bitwise_and(lanes + 16 * lg, 3) + 4 * r for r, lg in
             zip(rows_l, range(8))]

    def transpose_block(sl):
        @plsc.parallel_loop(0, _EMBED, unroll=4)
        def e_body(e):
            es4 = jnp.full((16,), 4 * e, dtype=jnp.int32)
            for lg in range(8):
                x = in_v[sl, e, pl.ds(16 * lg, 16)]
                cols = lax.bitwise_and(rot_l[lg] + es4, 127)
                plsc.store_scatter(out_v.at[sl], [rows_l[lg], cols], x)

    b00 = w * _MAIN_PW
    fire_in(b00, 0)

    def pair_body(i, carry):
        for sl in range(2):
            b = b00 + 2 * i + sl
            nxt = b + 1

            @pl.when(nxt < b00 + _MAIN_PW)
            def _():
                fire_in(nxt, 1 - sl)

            wait_in(b, sl)

            @pl.when(i > 0)
            def _():
                wait_out(b, sl)

            transpose_block(sl)
            fire_out(b, sl)
        return carry

    lax.fori_loop(0, _MAIN_PW // 2, pair_body, 0)
    wait_out(0, 0)
    wait_out(0, 1)

    @pl.when(w < 4)
    def _():
        b = _FULL_TILES - 4 + w
        pltpu.sync_copy(tokt_hbm.at[:, pl.ds(b * 128, 128)], in_v.at[0])
        transpose_block(0)
        pltpu.sync_copy(out_v.at[0], out_hbm.at[pl.ds(b * 32, 32)])

    @pl.when(w == 4)
    def _():
        pltpu.sync_copy(tail_hbm, out_hbm.at[pl.ds(_FULL_TILES * 32, 16)])


@functools.lru_cache(maxsize=None)
def _build_lookup(batch):
    @functools.partial(
        pl.kernel,
        mesh=_MESH,
        out_type=jax.ShapeDtypeStruct((_MAXLEN, _EMBED, batch), jnp.float32),
        scratch_types=[
            pltpu.VMEM((_MAXLEN, _STRIPE), jnp.int32),
            pltpu.VMEM((50, 128), jnp.float32),
            pltpu.VMEM((4, _STRIPE), jnp.int32),
            pltpu.VMEM((4, _STRIPE, 128), jnp.float32),
            pltpu.VMEM((2, _EMBED, _STRIPE), jnp.float32),
            pltpu.SemaphoreType.DMA,
            pltpu.SemaphoreType.DMA,
            pltpu.SemaphoreType.DMA,
            pltpu.SemaphoreType.DMA,
            pltpu.SemaphoreType.DMA,
            pltpu.SemaphoreType.DMA,
        ],
        compiler_params=_PARAMS,
    )
    def kern(idx_hbm, tok_hbm, pos_hbm, out_hbm,
             idx_v, pos_v, iv_v, g_v, o_v, gs0, gs1, gs2, gs3, ss0, ss1):
        w = _wid()
        b0 = w * _STRIPE
        pltpu.sync_copy(idx_hbm.at[:, pl.ds(b0, _STRIPE)], idx_v)
        pltpu.sync_copy(pos_hbm, pos_v)
        lanes = lax.iota(jnp.int32, 16)
        gsem = (gs0, gs1, gs2, gs3)
        ssem = (ss0, ss1)

        def fire_gather(t, sl):
            for g in range(8):
                ids = idx_v[t, pl.ds(16 * g, 16)]
                iv_v[sl, pl.ds(16 * g, 16)] = lax.shift_right_logical(ids, 2)
            pltpu.async_copy(tok_hbm.at[iv_v.at[sl]], g_v.at[sl], gsem[sl])

        def wait_gather(sl):
            pltpu.make_async_copy(
                tok_hbm.at[iv_v.at[sl]], g_v.at[sl], gsem[sl]).wait()

        def fire_store(t, sl):
            pltpu.async_copy(
                o_v.at[sl], out_hbm.at[t, :, pl.ds(b0, _STRIPE)], ssem[sl])

        def wait_store(t, sl):
            pltpu.make_async_copy(
                o_v.at[sl], out_hbm.at[t, :, pl.ds(b0, _STRIPE)], ssem[sl]).wait()

        def compute(t, sl, osl):
            cbs = []
            for g in range(8):
                ids = idx_v[t, pl.ds(16 * g, 16)]
                cbs.append(lax.bitwise_and(ids, 3) + 4 * lax.bitwise_and(
                    lax.shift_right_logical(ids, 2), 31))
            q16 = jnp.full((16,), lax.div(t * _EMBED, 128), dtype=jnp.int32)
            r0 = lax.rem(t * _EMBED, 128)

            @plsc.parallel_loop(0, _EMBED, unroll=4)
            def e_body(e):
                ps = plsc.load_gather(
                    pos_v, [q16, jnp.full((16,), r0 + e, dtype=jnp.int32)])
                for g in range(8):
                    cols = lax.bitwise_and(cbs[g] + 4 * e, 127)
                    vals = plsc.load_gather(
                        g_v.at[sl], [lanes + 16 * g, cols])
                    o_v[osl, e, pl.ds(16 * g, 16)] = vals + ps

        fire_gather(0, 0)
        fire_gather(1, 1)

        def quad_body(i, carry):
            for sl in range(4):
                t = 4 * i + sl
                osl = sl % 2

                @pl.when(t + 2 < _MAXLEN)
                def _():
                    fire_gather(t + 2, (sl + 2) % 4)

                wait_gather(sl)
                if sl >= 2:
                    wait_store(t, osl)
                else:
                    @pl.when(i > 0)
                    def _():
                        wait_store(t, osl)

                compute(t, sl, osl)
                fire_store(t, osl)
            return carry

        lax.fori_loop(0, _MAXLEN // 4, quad_body, 0)
        wait_store(_MAXLEN - 2, 0)
        wait_store(_MAXLEN - 1, 1)

    return kern


def kernel(inputs, token_table, pos_table):
    batch, maxlen = inputs.shape
    idx_t = inputs.T.astype(jnp.int32)
    pos128 = pos_table.reshape(50, 128)
    tail = (token_table[_FULL_TILES * 128:, :]
            .reshape(16, 4, _EMBED).transpose(0, 2, 1).reshape(16, 128))
    tail16 = jax.vmap(jnp.roll)(tail, 4 * jnp.arange(16))
    tok_lines = _pack_table(token_table.T, tail16)
    out = _build_lookup(batch)(idx_t, tok_lines, pos128)
    return jnp.transpose(out, (2, 0, 1))


# speedup vs baseline: 1.1374x; 1.1374x over previous
import functools

import jax
import jax.numpy as jnp
from jax import lax
from jax.experimental import pallas as pl
from jax.experimental.pallas import tpu as pltpu
from jax.experimental.pallas import tpu_sc as plsc

_MAXLEN = 200
_EMBED = 32
_NW = 32
_STRIPE = 128
_VOCAB = 1_000_000
_NLINE = _VOCAB // 4
_FULL_TILES = _VOCAB // 128
_MAIN_PW = _FULL_TILES // _NW

_PARAMS = pltpu.CompilerParams(
    use_tc_tiling_on_sc=True, needs_layout_passes=False)
_MESH = plsc.VectorSubcoreMesh(core_axis_name="c", subcore_axis_name="s")


def _wid():
    return lax.axis_index("s") * 2 + lax.axis_index("c")


@functools.partial(
    pl.kernel,
    mesh=_MESH,
    out_type=jax.ShapeDtypeStruct((_NLINE, 128), jnp.float32),
    scratch_types=[
        pltpu.VMEM((2, _EMBED, 128), jnp.float32),
        pltpu.VMEM((2, _EMBED, 128), jnp.float32),
        pltpu.SemaphoreType.DMA,
        pltpu.SemaphoreType.DMA,
        pltpu.SemaphoreType.DMA,
        pltpu.SemaphoreType.DMA,
    ],
    compiler_params=_PARAMS,
)
def _pack_table(tokt_hbm, tail_hbm, out_hbm, in_v, out_v, gs0, gs1, ss0, ss1):
    w = _wid()
    lanes = lax.iota(jnp.int32, 16)
    gsem = (gs0, gs1)
    ssem = (ss0, ss1)

    def fire_in(b, sl):
        pltpu.async_copy(
            tokt_hbm.at[:, pl.ds(b * 128, 128)], in_v.at[sl], gsem[sl])

    def wait_in(b, sl):
        pltpu.make_async_copy(
            tokt_hbm.at[:, pl.ds(b * 128, 128)], in_v.at[sl], gsem[sl]).wait()

    def fire_out(b, sl):
        pltpu.async_copy(
            out_v.at[sl], out_hbm.at[pl.ds(b * 32, 32)], ssem[sl])

    def wait_out(b, sl):
        pltpu.make_async_copy(
            out_v.at[sl], out_hbm.at[pl.ds(b * 32, 32)], ssem[sl]).wait()

    rows_l = [lax.shift_right_logical(lanes + 16 * lg, 2) for lg in range(8)]
    cola_l = [lax.bitwise_and(lanes + 16 * lg, 3) for lg in range(8)]

    def transpose_block(sl):
        @plsc.parallel_loop(0, _EMBED, unroll=4)
        def e_body(e):
            es4 = jnp.full((16,), 4 * e, dtype=jnp.int32)
            for lg in range(8):
                x = in_v[sl, e, pl.ds(16 * lg, 16)]
                plsc.store_scatter(
                    out_v.at[sl], [rows_l[lg], cola_l[lg] + es4], x)

    b00 = w * _MAIN_PW
    fire_in(b00, 0)

    def pair_body(i, carry):
        for sl in range(2):
            b = b00 + 2 * i + sl
            nxt = b + 1

            @pl.when(nxt < b00 + _MAIN_PW)
            def _():
                fire_in(nxt, 1 - sl)

            wait_in(b, sl)

            @pl.when(i > 0)
            def _():
                wait_out(b, sl)

            transpose_block(sl)
            fire_out(b, sl)
        return carry

    lax.fori_loop(0, _MAIN_PW // 2, pair_body, 0)
    wait_out(0, 0)
    wait_out(0, 1)

    @pl.when(w < 4)
    def _():
        b = _FULL_TILES - 4 + w
        pltpu.sync_copy(tokt_hbm.at[:, pl.ds(b * 128, 128)], in_v.at[0])
        transpose_block(0)
        pltpu.sync_copy(out_v.at[0], out_hbm.at[pl.ds(b * 32, 32)])

    @pl.when(w == 4)
    def _():
        pltpu.sync_copy(tail_hbm, out_hbm.at[pl.ds(_FULL_TILES * 32, 16)])


@functools.lru_cache(maxsize=None)
def _build_lookup(batch):
    @functools.partial(
        pl.kernel,
        mesh=_MESH,
        out_type=jax.ShapeDtypeStruct((_MAXLEN, _EMBED, batch), jnp.float32),
        scratch_types=[
            pltpu.VMEM((_MAXLEN, _STRIPE), jnp.int32),
            pltpu.VMEM((50, 128), jnp.float32),
            pltpu.VMEM((4, _STRIPE), jnp.int32),
            pltpu.VMEM((4, _STRIPE, 128), jnp.float32),
            pltpu.VMEM((2, _EMBED, _STRIPE), jnp.float32),
            pltpu.SemaphoreType.DMA,
            pltpu.SemaphoreType.DMA,
            pltpu.SemaphoreType.DMA,
            pltpu.SemaphoreType.DMA,
            pltpu.SemaphoreType.DMA,
            pltpu.SemaphoreType.DMA,
        ],
        compiler_params=_PARAMS,
    )
    def kern(idx_hbm, tok_hbm, pos_hbm, out_hbm,
             idx_v, pos_v, iv_v, g_v, o_v, gs0, gs1, gs2, gs3, ss0, ss1):
        w = _wid()
        b0 = w * _STRIPE
        pltpu.sync_copy(idx_hbm.at[:, pl.ds(b0, _STRIPE)], idx_v)
        pltpu.sync_copy(pos_hbm, pos_v)
        lanes = lax.iota(jnp.int32, 16)
        gsem = (gs0, gs1, gs2, gs3)
        ssem = (ss0, ss1)

        def fire_gather(t, sl):
            for g in range(8):
                ids = idx_v[t, pl.ds(16 * g, 16)]
                iv_v[sl, pl.ds(16 * g, 16)] = lax.shift_right_logical(ids, 2)
            pltpu.async_copy(tok_hbm.at[iv_v.at[sl]], g_v.at[sl], gsem[sl])

        def wait_gather(sl):
            pltpu.make_async_copy(
                tok_hbm.at[iv_v.at[sl]], g_v.at[sl], gsem[sl]).wait()

        def fire_store(t, sl):
            pltpu.async_copy(
                o_v.at[sl], out_hbm.at[t, :, pl.ds(b0, _STRIPE)], ssem[sl])

        def wait_store(t, sl):
            pltpu.make_async_copy(
                o_v.at[sl], out_hbm.at[t, :, pl.ds(b0, _STRIPE)], ssem[sl]).wait()

        def compute(t, sl, osl):
            cbs = []
            for g in range(8):
                ids = idx_v[t, pl.ds(16 * g, 16)]
                cbs.append(lax.bitwise_and(ids, 3))
            q16 = jnp.full((16,), lax.div(t * _EMBED, 128), dtype=jnp.int32)
            r0 = lax.rem(t * _EMBED, 128)

            @plsc.parallel_loop(0, _EMBED, unroll=4)
            def e_body(e):
                ps = plsc.load_gather(
                    pos_v, [q16, jnp.full((16,), r0 + e, dtype=jnp.int32)])
                for g in range(8):
                    vals = plsc.load_gather(
                        g_v.at[sl], [lanes + 16 * g, cbs[g] + 4 * e])
                    o_v[osl, e, pl.ds(16 * g, 16)] = vals + ps

        fire_gather(0, 0)
        fire_gather(1, 1)

        def quad_body(i, carry):
            for sl in range(4):
                t = 4 * i + sl
                osl = sl % 2

                @pl.when(t + 2 < _MAXLEN)
                def _():
                    fire_gather(t + 2, (sl + 2) % 4)

                wait_gather(sl)
                if sl >= 2:
                    wait_store(t, osl)
                else:
                    @pl.when(i > 0)
                    def _():
                        wait_store(t, osl)

                compute(t, sl, osl)
                fire_store(t, osl)
            return carry

        lax.fori_loop(0, _MAXLEN // 4, quad_body, 0)
        wait_store(_MAXLEN - 2, 0)
        wait_store(_MAXLEN - 1, 1)

    return kern


def kernel(inputs, token_table, pos_table):
    batch, maxlen = inputs.shape
    idx_t = inputs.T.astype(jnp.int32)
    pos128 = pos_table.reshape(50, 128)
    tail16 = (token_table[_FULL_TILES * 128:, :]
              .reshape(16, 4, _EMBED).transpose(0, 2, 1).reshape(16, 128))
    tok_lines = _pack_table(token_table.T, tail16)
    out = _build_lookup(batch)(idx_t, tok_lines, pos128)
    return jnp.transpose(out, (2, 0, 1))
